# Initial kernel scaffold; baseline (speedup 1.0000x reference)
import jax
import jax.numpy as jnp
from jax.experimental import pallas as pl


def kernel(x, batch, W_ih, W_hh, b_ih, b_hh):
    raise NotImplementedError("write your pallas kernel here")


# speedup vs baseline: 6.0868x; 6.0868x over previous
import functools

import jax
import jax.numpy as jnp
from jax import lax
from jax.experimental import pallas as pl
from jax.experimental.pallas import tpu as pltpu
from jax.experimental.pallas import tpu_sc as plsc

N = 100000
C = 128
B = 512
T = 4

NC = 2
NS = 16
NW = NC * NS
NP = 3200
N_PAD = NW * NP
CH = 128
NCHUNK = NP // CH
SEG = 640
RW = 144
NEG = -1e30

_mesh = plsc.VectorSubcoreMesh(core_axis_name="c", subcore_axis_name="s")

_GDN = lax.GatherDimensionNumbers(
    offset_dims=(), collapsed_slice_dims=(0,), start_index_map=(0,))


def _shuffle16(v, idx):
    return lax.gather(v, idx[:, None], _GDN, (1,),
                      mode=lax.GatherScatterMode.PROMISE_IN_BOUNDS)


def _hsum16(v, lanes):
    for sh in (8, 4, 2, 1):
        v = v + _shuffle16(v, lanes ^ sh)
    return v


def _p1_body(x_hbm, b_hbm, h_hbm, e_hbm, mp_hbm, xbuf, bbuf, hbuf, ebuf, mvec,
             mloc):
    cid = lax.axis_index("c")
    sid = lax.axis_index("s")
    wid = cid * NS + sid
    base = wid * NP

    pltpu.sync_copy(h_hbm, hbuf)

    def minit(i, carry):
        mloc[i] = NEG
        return carry

    lax.fori_loop(0, SEG, minit, 0)

    lanes = lax.iota(jnp.int32, 16)

    def chunk_body(cc, carry):
        off = base + cc * CH
        pltpu.sync_copy(x_hbm.at[pl.ds(off, CH), :], xbuf)
        pltpu.sync_copy(b_hbm.at[pl.ds(off, CH)], bbuf)

        def group_body(j, c2):
            bv = bbuf[pl.ds(j * 16, 16)]
            evec = jnp.zeros((16,), jnp.float32)
            for l in range(16):
                s = bv[l]
                i = j * 16 + l
                acc = xbuf[i, pl.ds(0, 16)] * hbuf[s, pl.ds(0, 16)]
                for k in range(1, 8):
                    acc = acc + xbuf[i, pl.ds(k * 16, 16)] * hbuf[s, pl.ds(k * 16, 16)]
                sv = _hsum16(acc, lanes)
                evec = jnp.where(lanes == l, sv, evec)
                mloc[s] = jnp.maximum(mloc[s], sv[0])
            ebuf[pl.ds(j * 16, 16)] = evec
            return c2

        lax.fori_loop(0, CH // 16, group_body, 0)
        pltpu.sync_copy(ebuf, e_hbm.at[pl.ds(off, CH)])
        return carry

    lax.fori_loop(0, NCHUNK, chunk_body, 0)

    def pack_body(j, carry):
        v = jnp.zeros((16,), jnp.float32)
        for l in range(16):
            v = jnp.where(lanes == l, mloc[j * 16 + l], v)
        mvec[pl.ds(j * 16, 16)] = v
        return carry

    lax.fori_loop(0, SEG // 16, pack_body, 0)
    pltpu.sync_copy(mvec, mp_hbm.at[wid])


@functools.partial(
    pl.kernel,
    mesh=_mesh,
    out_type=[
        jax.ShapeDtypeStruct((N_PAD,), jnp.float32),
        jax.ShapeDtypeStruct((NW, SEG), jnp.float32),
    ],
    scratch_types=[
        pltpu.VMEM((CH, C), jnp.float32),
        pltpu.VMEM((CH,), jnp.int32),
        pltpu.VMEM((B, C), jnp.float32),
        pltpu.VMEM((CH,), jnp.float32),
        pltpu.VMEM((SEG,), jnp.float32),
        pltpu.SMEM((SEG,), jnp.float32),
    ],
)
def _sc_p1(x_hbm, b_hbm, h_hbm, e_hbm, mp_hbm, xbuf, bbuf, hbuf, ebuf, mvec,
           mloc):
    _p1_body(x_hbm, b_hbm, h_hbm, e_hbm, mp_hbm, xbuf, bbuf, hbuf, ebuf, mvec,
             mloc)


def _p2_body(x_hbm, b_hbm, e_hbm, mp_hbm, rtab_hbm, dp_hbm,
             xbuf, bbuf, ebuf, dvec, m, dloc, mpbuf, stage, table):
    cid = lax.axis_index("c")
    sid = lax.axis_index("s")
    wid = cid * NS + sid
    base = wid * NP

    def dinit(i, carry):
        dloc[i] = 0.0
        return carry

    lax.fori_loop(0, SEG, dinit, 0)

    pltpu.sync_copy(mp_hbm, mpbuf)

    def red_body(j, carry):
        acc = mpbuf[0, pl.ds(j * 16, 16)]

        def w_body(w, a):
            return jnp.maximum(a, mpbuf[w, pl.ds(j * 16, 16)])

        acc = lax.fori_loop(1, NW, w_body, acc)
        for l in range(16):
            m[j * 16 + l] = acc[l]
        return carry

    lax.fori_loop(0, SEG // 16, red_body, 0)

    @pl.when(sid == 0)
    def _zero():
        def zrow(i, carry):
            for k in range(C // 16):
                stage[i, pl.ds(k * 16, 16)] = jnp.zeros((16,), jnp.float32)
            return carry

        lax.fori_loop(0, CH, zrow, 0)
        for blk in range(SEG // CH):
            pltpu.sync_copy(stage, table.at[pl.ds(blk * CH, CH), :])

    plsc.subcore_barrier()

    lanes = lax.iota(jnp.int32, 16)

    def chunk_body(cc, carry):
        off = base + cc * CH
        pltpu.sync_copy(x_hbm.at[pl.ds(off, CH), :], xbuf)
        pltpu.sync_copy(b_hbm.at[pl.ds(off, CH)], bbuf)
        pltpu.sync_copy(e_hbm.at[pl.ds(off, CH)], ebuf)

        def group_body(j, c2):
            ev = ebuf[pl.ds(j * 16, 16)]
            bv = bbuf[pl.ds(j * 16, 16)]
            mv = jnp.zeros((16,), jnp.float32)
            for l in range(16):
                mv = jnp.where(lanes == l, m[bv[l]], mv)
            wv = jnp.exp(ev - mv)
            for l in range(16):
                w = wv[l]
                i = j * 16 + l
                for k in range(8):
                    stage[i, pl.ds(k * 16, 16)] = xbuf[i, pl.ds(k * 16, 16)] * w
                s = bv[l]
                dloc[s] = dloc[s] + w
            return c2

        lax.fori_loop(0, CH // 16, group_body, 0)
        pltpu.sync_copy(stage, table.at[bbuf], add=True)
        return carry

    lax.fori_loop(0, NCHUNK, chunk_body, 0)

    def dpack(j, carry):
        v = jnp.zeros((16,), jnp.float32)
        for l in range(16):
            v = jnp.where(lanes == l, dloc[j * 16 + l], v)
        dvec[pl.ds(j * 16, 16)] = v
        return carry

    lax.fori_loop(0, SEG // 16, dpack, 0)
    pltpu.sync_copy(dvec, dp_hbm.at[wid])

    plsc.subcore_barrier()

    @pl.when(sid == 0)
    def _writeout():
        pltpu.sync_copy(table, rtab_hbm.at[cid])


@functools.partial(
    pl.kernel,
    mesh=_mesh,
    out_type=[
        jax.ShapeDtypeStruct((NC, SEG, C), jnp.float32),
        jax.ShapeDtypeStruct((NW, SEG), jnp.float32),
    ],
    scratch_types=[
        pltpu.VMEM((CH, C), jnp.float32),
        pltpu.VMEM((CH,), jnp.int32),
        pltpu.VMEM((CH,), jnp.float32),
        pltpu.VMEM((SEG,), jnp.float32),
        pltpu.SMEM((SEG,), jnp.float32),
        pltpu.SMEM((SEG,), jnp.float32),
        pltpu.VMEM((NW, SEG), jnp.float32),
        pltpu.VMEM((CH, C), jnp.float32),
        pltpu.VMEM_SHARED((SEG, C), jnp.float32),
    ],
)
def _sc_p2(x_hbm, b_hbm, e_hbm, mp_hbm, rtab_hbm, dp_hbm,
           xbuf, bbuf, ebuf, dvec, m, dloc, mpbuf, stage, table):
    _p2_body(x_hbm, b_hbm, e_hbm, mp_hbm, rtab_hbm, dp_hbm,
             xbuf, bbuf, ebuf, dvec, m, dloc, mpbuf, stage, table)


def _lstm_tc_body(q_ref, h_ref, c_ref, wih_ref, whh_ref, b_ref, ho_ref, co_ref):
    gates = (
        lax.dot_general(q_ref[...], wih_ref[...],
                        (((1,), (1,)), ((), ())),
                        preferred_element_type=jnp.float32)
        + lax.dot_general(h_ref[...], whh_ref[...],
                          (((1,), (1,)), ((), ())),
                          preferred_element_type=jnp.float32)
        + b_ref[...]
    )
    i_g = jax.nn.sigmoid(gates[:, 0 * C:1 * C])
    f_g = jax.nn.sigmoid(gates[:, 1 * C:2 * C])
    g_g = jnp.tanh(gates[:, 2 * C:3 * C])
    o_g = jax.nn.sigmoid(gates[:, 3 * C:4 * C])
    c_new = f_g * c_ref[...] + i_g * g_g
    co_ref[...] = c_new
    ho_ref[...] = o_g * jnp.tanh(c_new)


_lstm_tc = pl.pallas_call(
    _lstm_tc_body,
    out_shape=[
        jax.ShapeDtypeStruct((B, C), jnp.float32),
        jax.ShapeDtypeStruct((B, C), jnp.float32),
    ],
)


def _asm_tc_body(h_ref, rtab_ref, dp_ref, q_ref):
    r = rtab_ref[0, :B, :] + rtab_ref[1, :B, :]
    d = jnp.sum(dp_ref[:, :B], axis=0)[:, None] + 1e-16
    q_ref[:, :C] = h_ref[...]
    q_ref[:, C:] = r / d


_asm_tc = pl.pallas_call(
    _asm_tc_body,
    out_shape=jax.ShapeDtypeStruct((B, 2 * C), jnp.float32),
)


def kernel(x, batch, W_ih, W_hh, b_ih, b_hh):
    xp = jnp.zeros((N_PAD, C), jnp.float32).at[:N].set(x)
    bp = jnp.full((N_PAD,), B, jnp.int32).at[:N].set(batch.astype(jnp.int32))
    bias = (b_ih + b_hh).reshape(1, 4 * C)

    h = jnp.zeros((B, C), jnp.float32)
    c = jnp.zeros((B, C), jnp.float32)
    q_star = jnp.zeros((B, 2 * C), jnp.float32)
    for _ in range(T):
        h, c = _lstm_tc(q_star, h, c, W_ih, W_hh, bias)
        e, mpart = _sc_p1(xp, bp, h)
        rtab, dpart = _sc_p2(xp, bp, e, mpart)
        q_star = _asm_tc(h, rtab, dpart)
    return q_star
